# baseline (speedup 1.0000x reference)
import functools

import jax
import jax.numpy as jnp
from jax import lax
from jax.experimental import pallas as pl
from jax.experimental.pallas import tpu as pltpu
from jax.experimental.pallas import tpu_sc as plsc

_NUM_CORES = 2
_NUM_SUBCORES = 16
_NW = _NUM_CORES * _NUM_SUBCORES
_LANES = 16

_CHUNK = 128


def _sqrt16(a):
    ai = lax.bitcast_convert_type(a, jnp.int32)
    yi = jnp.int32(0x5F3759DF) - (ai >> 1)
    y = lax.bitcast_convert_type(yi, jnp.float32)
    for _ in range(2):
        y = y * (1.5 - 0.5 * a * y * y)
    return a * y


_NBUF = 4


def _edge_score_body(x_hbm, ei_hbm, ni_hbm, *refs,
                     e_real, e_per_w, n_pad, c_chunks):
    row_bufs = refs[0:_NBUF]
    col_bufs = refs[_NBUF:2 * _NBUF]
    a_bufs = refs[2 * _NBUF:3 * _NBUF]
    b_bufs = refs[3 * _NBUF:4 * _NBUF]
    t_v = refs[4 * _NBUF]
    ni_v = refs[4 * _NBUF + 1]
    a_sems = refs[4 * _NBUF + 2:5 * _NBUF + 2]
    b_sems = refs[5 * _NBUF + 2:6 * _NBUF + 2]
    sid = lax.axis_index("s")
    cid = lax.axis_index("c")
    wid = sid * _NUM_CORES + cid
    pair_chunks = e_per_w // _CHUNK
    c0 = max(_NBUF, (int(pair_chunks * 0.60) // _NBUF) * _NBUF)
    c1 = pair_chunks - c0
    num_chunks = jnp.where(cid == 0, c0, c1)
    base = sid * e_per_w + cid * (c0 * _CHUNK)

    zeros16 = jnp.zeros((_LANES,), jnp.float32)

    def _zero(i, carry):
        ni_v[pl.ds(i * _LANES, _LANES)] = zeros16
        return carry

    lax.fori_loop(0, n_pad // _LANES, _zero, 0)

    def _load_idx(ch, buf):
        off = base + ch * _CHUNK
        pltpu.sync_copy(ei_hbm.at[0, pl.ds(off, _CHUNK)], row_bufs[buf])
        pltpu.sync_copy(ei_hbm.at[1, pl.ds(off, _CHUNK)], col_bufs[buf])

    def _issue(buf):
        cp_a = pltpu.async_copy(x_hbm.at[row_bufs[buf]], a_bufs[buf],
                                a_sems[buf])
        cp_b = pltpu.async_copy(x_hbm.at[col_bufs[buf]], b_bufs[buf],
                                b_sems[buf])
        return cp_a, cp_b

    def _compute(ch, buf):
        a_v = a_bufs[buf]
        b_v = b_bufs[buf]
        off = base + ch * _CHUNK

        def _group_body(g, gcarry):
            gbase = g * _LANES
            lanes = lax.iota(jnp.int32, _LANES)
            for e in range(_LANES):
                r = gbase + e
                acc = jnp.zeros((2 * _LANES,), jnp.bfloat16)
                for c in range(c_chunks):
                    aw = a_v[r, pl.ds(c * _LANES, _LANES)]
                    bw = b_v[r, pl.ds(c * _LANES, _LANES)]
                    av = plsc.bitcast(aw, jnp.bfloat16)
                    bv = plsc.bitcast(bw, jnp.bfloat16)
                    d = av - bv
                    acc = acc + d * d
                lo, hi = plsc.unpack(acc, format=plsc.PackFormat.INTERLEAVED)
                t_v[pl.ds(e * _LANES, _LANES)] = lo + hi
            colidx = lanes * _LANES
            dsq = jnp.zeros((_LANES,), jnp.float32)
            for c in range(_LANES):
                dsq = dsq + plsc.load_gather(t_v, [colidx + c])
            dist = _sqrt16(dsq + 1e-6)
            gid = off + gbase + lanes
            mask = gid < e_real
            ridx = row_bufs[buf][pl.ds(gbase, _LANES)]
            plsc.addupdate_scatter(ni_v, [ridx], dist, mask=mask)
            return gcarry

        lax.fori_loop(0, _CHUNK // _LANES, _group_body, 0)

    def _outer(it, carry):
        copies = []
        for buf in range(_NBUF):
            _load_idx(it * _NBUF + buf, buf)
            copies.append(_issue(buf))
        for buf in range(_NBUF):
            cp_a, cp_b = copies[buf]
            cp_a.wait()
            cp_b.wait()
            _compute(it * _NBUF + buf, buf)
        return carry

    lax.fori_loop(0, num_chunks // _NBUF, _outer, 0)

    pltpu.sync_copy(ni_v, ni_hbm.at[wid])


def _sc_edge_scores(xh, ei_pad, *, e_real, n_pad):
    n, c = xh.shape
    e_per_w = ei_pad.shape[1] // _NUM_SUBCORES
    mesh = plsc.VectorSubcoreMesh(
        core_axis_name="c", subcore_axis_name="s",
        num_cores=_NUM_CORES, num_subcores=_NUM_SUBCORES)
    scratch = (
        [pltpu.VMEM((_CHUNK,), jnp.int32)] * (2 * _NBUF)
        + [pltpu.VMEM((_CHUNK, c), jnp.int32)] * (2 * _NBUF)
        + [pltpu.VMEM((_LANES * _LANES,), jnp.float32),
           pltpu.VMEM((n_pad,), jnp.float32)]
        + [pltpu.SemaphoreType.DMA] * (2 * _NBUF)
    )
    kfn = pl.kernel(
        functools.partial(_edge_score_body, e_real=e_real, e_per_w=e_per_w,
                          n_pad=n_pad, c_chunks=c // _LANES),
        out_type=jax.ShapeDtypeStruct((_NW, n_pad), jnp.float32),
        mesh=mesh,
        compiler_params=pltpu.CompilerParams(needs_layout_passes=False,
                                             use_tc_tiling_on_sc=False),
        scratch_types=scratch,
    )
    return kfn(xh, ei_pad)


def _pool_body(xe_ref, xo_ref, nie_ref, nio_ref, out_ref):
    we = jax.nn.sigmoid(jnp.sum(nie_ref[...], axis=1, keepdims=True))
    wo = jax.nn.sigmoid(jnp.sum(nio_ref[...], axis=1, keepdims=True))
    out_ref[...] = 0.5 * (xe_ref[...] * we + xo_ref[...] * wo)


def _tc_pool(xe, xo, nie_t, nio_t):
    k, c = xe.shape
    blk = 1000
    grid = k // blk
    return pl.pallas_call(
        _pool_body,
        grid=(grid,),
        in_specs=[
            pl.BlockSpec((blk, c), lambda i: (i, 0)),
            pl.BlockSpec((blk, c), lambda i: (i, 0)),
            pl.BlockSpec((blk, _NW), lambda i: (i, 0)),
            pl.BlockSpec((blk, _NW), lambda i: (i, 0)),
        ],
        out_specs=pl.BlockSpec((blk, c), lambda i: (i, 0)),
        out_shape=jax.ShapeDtypeStruct((k, c), jnp.float32),
    )(xe, xo, nie_t, nio_t)


def kernel(x, edge_index):
    b, n, c = x.shape
    e = edge_index.shape[1]
    num_keep = max(1, int(n * 0.5))

    x2d = x.reshape(n, c)

    tile_e = _NW * _CHUNK * _NBUF
    e_pad = ((e + tile_e - 1) // tile_e) * tile_e
    ei_pad = jnp.pad(edge_index.astype(jnp.int32), ((0, 0), (0, e_pad - e)))
    n_pad = ((n + _LANES - 1) // _LANES) * _LANES

    xh = jax.lax.bitcast_convert_type(
        x2d.astype(jnp.bfloat16).reshape(n, c // 2, 2), jnp.int32)
    nip = _sc_edge_scores(xh, ei_pad, e_real=e, n_pad=n_pad)

    ni = nip[:, :n]
    nie_t = ni[:, 0::2].T
    nio_t = ni[:, 1::2].T
    xe = x2d[0::2]
    xo = x2d[1::2]

    pooled = _tc_pool(xe, xo, nie_t, nio_t)
    x_pooled = pooled.reshape(b, num_keep, c)

    idx = jnp.arange(num_keep)
    left = idx[:-1]
    right = idx[1:]
    new_edge_index = jnp.concatenate(
        [jnp.stack([left, right], axis=0), jnp.stack([right, left], axis=0)],
        axis=1)
    return x_pooled, new_edge_index


# speedup vs baseline: 1.0325x; 1.0325x over previous
import functools

import jax
import jax.numpy as jnp
from jax import lax
from jax.experimental import pallas as pl
from jax.experimental.pallas import tpu as pltpu
from jax.experimental.pallas import tpu_sc as plsc

_NUM_CORES = 2
_NUM_SUBCORES = 16
_NW = _NUM_CORES * _NUM_SUBCORES
_LANES = 16

_CHUNK = 128


def _sqrt16(a):
    ai = lax.bitcast_convert_type(a, jnp.int32)
    yi = jnp.int32(0x5F3759DF) - (ai >> 1)
    y = lax.bitcast_convert_type(yi, jnp.float32)
    for _ in range(2):
        y = y * (1.5 - 0.5 * a * y * y)
    return a * y


_NBUF = 4


def _edge_score_body(x_hbm, ei_hbm, ni_hbm, *refs,
                     e_real, e_per_w, n_pad, c_chunks):
    row_bufs = refs[0:_NBUF]
    col_bufs = refs[_NBUF:2 * _NBUF]
    a_bufs = refs[2 * _NBUF:3 * _NBUF]
    b_bufs = refs[3 * _NBUF:4 * _NBUF]
    t_v = refs[4 * _NBUF]
    ni_v = refs[4 * _NBUF + 1]
    a_sems = refs[4 * _NBUF + 2:5 * _NBUF + 2]
    b_sems = refs[5 * _NBUF + 2:6 * _NBUF + 2]
    sid = lax.axis_index("s")
    cid = lax.axis_index("c")
    wid = sid * _NUM_CORES + cid
    pair_chunks = e_per_w // _CHUNK
    c0 = max(_NBUF, (int(pair_chunks * 0.65) // _NBUF) * _NBUF)
    c1 = pair_chunks - c0
    num_chunks = jnp.where(cid == 0, c0, c1)
    base = sid * e_per_w + cid * (c0 * _CHUNK)

    zeros16 = jnp.zeros((_LANES,), jnp.float32)

    def _zero(i, carry):
        ni_v[pl.ds(i * _LANES, _LANES)] = zeros16
        return carry

    lax.fori_loop(0, n_pad // _LANES, _zero, 0)

    def _load_idx(ch, buf):
        off = base + ch * _CHUNK
        pltpu.sync_copy(ei_hbm.at[0, pl.ds(off, _CHUNK)], row_bufs[buf])
        pltpu.sync_copy(ei_hbm.at[1, pl.ds(off, _CHUNK)], col_bufs[buf])

    def _issue(buf):
        cp_a = pltpu.async_copy(x_hbm.at[row_bufs[buf]], a_bufs[buf],
                                a_sems[buf])
        cp_b = pltpu.async_copy(x_hbm.at[col_bufs[buf]], b_bufs[buf],
                                b_sems[buf])
        return cp_a, cp_b

    def _compute(ch, buf):
        a_v = a_bufs[buf]
        b_v = b_bufs[buf]
        off = base + ch * _CHUNK

        def _group_body(g, gcarry):
            gbase = g * _LANES
            lanes = lax.iota(jnp.int32, _LANES)
            for e in range(_LANES):
                r = gbase + e
                acc = jnp.zeros((2 * _LANES,), jnp.bfloat16)
                for c in range(c_chunks):
                    aw = a_v[r, pl.ds(c * _LANES, _LANES)]
                    bw = b_v[r, pl.ds(c * _LANES, _LANES)]
                    av = plsc.bitcast(aw, jnp.bfloat16)
                    bv = plsc.bitcast(bw, jnp.bfloat16)
                    d = av - bv
                    acc = acc + d * d
                lo, hi = plsc.unpack(acc, format=plsc.PackFormat.INTERLEAVED)
                t_v[pl.ds(e * _LANES, _LANES)] = lo + hi
            colidx = lanes * _LANES
            dsq = jnp.zeros((_LANES,), jnp.float32)
            for c in range(_LANES):
                dsq = dsq + plsc.load_gather(t_v, [colidx + c])
            dist = _sqrt16(dsq + 1e-6)
            gid = off + gbase + lanes
            mask = gid < e_real
            ridx = row_bufs[buf][pl.ds(gbase, _LANES)]
            plsc.addupdate_scatter(ni_v, [ridx], dist, mask=mask)
            return gcarry

        lax.fori_loop(0, _CHUNK // _LANES, _group_body, 0)

    def _outer(it, carry):
        copies = []
        for buf in range(_NBUF):
            _load_idx(it * _NBUF + buf, buf)
            copies.append(_issue(buf))
        for buf in range(_NBUF):
            cp_a, cp_b = copies[buf]
            cp_a.wait()
            cp_b.wait()
            _compute(it * _NBUF + buf, buf)
        return carry

    lax.fori_loop(0, num_chunks // _NBUF, _outer, 0)

    pltpu.sync_copy(ni_v, ni_hbm.at[wid])


def _sc_edge_scores(xh, ei_pad, *, e_real, n_pad):
    n, c = xh.shape
    e_per_w = ei_pad.shape[1] // _NUM_SUBCORES
    mesh = plsc.VectorSubcoreMesh(
        core_axis_name="c", subcore_axis_name="s",
        num_cores=_NUM_CORES, num_subcores=_NUM_SUBCORES)
    scratch = (
        [pltpu.VMEM((_CHUNK,), jnp.int32)] * (2 * _NBUF)
        + [pltpu.VMEM((_CHUNK, c), jnp.int32)] * (2 * _NBUF)
        + [pltpu.VMEM((_LANES * _LANES,), jnp.float32),
           pltpu.VMEM((n_pad,), jnp.float32)]
        + [pltpu.SemaphoreType.DMA] * (2 * _NBUF)
    )
    kfn = pl.kernel(
        functools.partial(_edge_score_body, e_real=e_real, e_per_w=e_per_w,
                          n_pad=n_pad, c_chunks=c // _LANES),
        out_type=jax.ShapeDtypeStruct((_NW, n_pad), jnp.float32),
        mesh=mesh,
        compiler_params=pltpu.CompilerParams(needs_layout_passes=False,
                                             use_tc_tiling_on_sc=False),
        scratch_types=scratch,
    )
    return kfn(xh, ei_pad)


def _pool_body(xe_ref, xo_ref, nie_ref, nio_ref, out_ref):
    we = jax.nn.sigmoid(jnp.sum(nie_ref[...], axis=1, keepdims=True))
    wo = jax.nn.sigmoid(jnp.sum(nio_ref[...], axis=1, keepdims=True))
    out_ref[...] = 0.5 * (xe_ref[...] * we + xo_ref[...] * wo)


def _tc_pool(xe, xo, nie_t, nio_t):
    k, c = xe.shape
    blk = 1000
    grid = k // blk
    return pl.pallas_call(
        _pool_body,
        grid=(grid,),
        in_specs=[
            pl.BlockSpec((blk, c), lambda i: (i, 0)),
            pl.BlockSpec((blk, c), lambda i: (i, 0)),
            pl.BlockSpec((blk, _NW), lambda i: (i, 0)),
            pl.BlockSpec((blk, _NW), lambda i: (i, 0)),
        ],
        out_specs=pl.BlockSpec((blk, c), lambda i: (i, 0)),
        out_shape=jax.ShapeDtypeStruct((k, c), jnp.float32),
    )(xe, xo, nie_t, nio_t)


def kernel(x, edge_index):
    b, n, c = x.shape
    e = edge_index.shape[1]
    num_keep = max(1, int(n * 0.5))

    x2d = x.reshape(n, c)

    tile_e = _NW * _CHUNK * _NBUF
    e_pad = ((e + tile_e - 1) // tile_e) * tile_e
    ei_pad = jnp.pad(edge_index.astype(jnp.int32), ((0, 0), (0, e_pad - e)))
    n_pad = ((n + _LANES - 1) // _LANES) * _LANES

    xh = jax.lax.bitcast_convert_type(
        x2d.astype(jnp.bfloat16).reshape(n, c // 2, 2), jnp.int32)
    nip = _sc_edge_scores(xh, ei_pad, e_real=e, n_pad=n_pad)

    ni = nip[:, :n]
    nie_t = ni[:, 0::2].T
    nio_t = ni[:, 1::2].T
    xe = x2d[0::2]
    xo = x2d[1::2]

    pooled = _tc_pool(xe, xo, nie_t, nio_t)
    x_pooled = pooled.reshape(b, num_keep, c)

    idx = jnp.arange(num_keep)
    left = idx[:-1]
    right = idx[1:]
    new_edge_index = jnp.concatenate(
        [jnp.stack([left, right], axis=0), jnp.stack([right, left], axis=0)],
        axis=1)
    return x_pooled, new_edge_index


# speedup vs baseline: 1.2909x; 1.2502x over previous
import functools

import jax
import jax.numpy as jnp
from jax import lax
from jax.experimental import pallas as pl
from jax.experimental.pallas import tpu as pltpu
from jax.experimental.pallas import tpu_sc as plsc

_NUM_CORES = 2
_NUM_SUBCORES = 16
_NW = _NUM_CORES * _NUM_SUBCORES
_LANES = 16

_CHUNK = 128


def _sqrt16(a):
    ai = lax.bitcast_convert_type(a, jnp.int32)
    yi = jnp.int32(0x5F3759DF) - (ai >> 1)
    y = lax.bitcast_convert_type(yi, jnp.float32)
    for _ in range(2):
        y = y * (1.5 - 0.5 * a * y * y)
    return a * y


_NBUF = 4


def _edge_score_body(x_hbm, ei_hbm, ni_hbm, *refs,
                     e_real, e_per_w, n_pad, c_chunks):
    row_bufs = refs[0:_NBUF]
    col_bufs = refs[_NBUF:2 * _NBUF]
    a_bufs = refs[2 * _NBUF:3 * _NBUF]
    b_bufs = refs[3 * _NBUF:4 * _NBUF]
    t_v = refs[4 * _NBUF]
    ni_v = refs[4 * _NBUF + 1]
    xs_ref = refs[4 * _NBUF + 2]
    a_sems = refs[4 * _NBUF + 3:5 * _NBUF + 3]
    b_sems = refs[5 * _NBUF + 3:6 * _NBUF + 3]
    sid = lax.axis_index("s")
    cid = lax.axis_index("c")
    wid = sid * _NUM_CORES + cid
    pair_chunks = e_per_w // _CHUNK
    c0 = max(_NBUF, (int(pair_chunks * 0.65) // _NBUF) * _NBUF)
    c1 = pair_chunks - c0
    num_chunks = jnp.where(cid == 0, c0, c1)
    base = sid * e_per_w + cid * (c0 * _CHUNK)

    zeros16 = jnp.zeros((_LANES,), jnp.float32)

    def _zero(i, carry):
        ni_v[pl.ds(i * _LANES, _LANES)] = zeros16
        return carry

    @pl.when(sid == 0)
    def _fill_shared():
        pltpu.sync_copy(x_hbm, xs_ref)

    plsc.subcore_barrier()

    lax.fori_loop(0, n_pad // _LANES, _zero, 0)

    def _load_idx(ch, buf):
        off = base + ch * _CHUNK
        pltpu.sync_copy(ei_hbm.at[0, pl.ds(off, _CHUNK)], row_bufs[buf])
        pltpu.sync_copy(ei_hbm.at[1, pl.ds(off, _CHUNK)], col_bufs[buf])

    def _issue(buf):
        cp_a = pltpu.async_copy(xs_ref.at[row_bufs[buf]], a_bufs[buf],
                                a_sems[buf])
        cp_b = pltpu.async_copy(xs_ref.at[col_bufs[buf]], b_bufs[buf],
                                b_sems[buf])
        return cp_a, cp_b

    def _compute(ch, buf):
        a_v = a_bufs[buf]
        b_v = b_bufs[buf]
        off = base + ch * _CHUNK

        def _group_body(g, gcarry):
            gbase = g * _LANES
            lanes = lax.iota(jnp.int32, _LANES)
            for e in range(_LANES):
                r = gbase + e
                acc = jnp.zeros((2 * _LANES,), jnp.bfloat16)
                for c in range(c_chunks):
                    aw = a_v[r, pl.ds(c * _LANES, _LANES)]
                    bw = b_v[r, pl.ds(c * _LANES, _LANES)]
                    av = plsc.bitcast(aw, jnp.bfloat16)
                    bv = plsc.bitcast(bw, jnp.bfloat16)
                    d = av - bv
                    acc = acc + d * d
                lo, hi = plsc.unpack(acc, format=plsc.PackFormat.INTERLEAVED)
                t_v[pl.ds(e * _LANES, _LANES)] = lo + hi
            colidx = lanes * _LANES
            dsq = jnp.zeros((_LANES,), jnp.float32)
            for c in range(_LANES):
                dsq = dsq + plsc.load_gather(t_v, [colidx + c])
            dist = _sqrt16(dsq + 1e-6)
            gid = off + gbase + lanes
            mask = gid < e_real
            ridx = row_bufs[buf][pl.ds(gbase, _LANES)]
            plsc.addupdate_scatter(ni_v, [ridx], dist, mask=mask)
            return gcarry

        lax.fori_loop(0, _CHUNK // _LANES, _group_body, 0)

    def _outer(it, carry):
        copies = []
        for buf in range(_NBUF):
            _load_idx(it * _NBUF + buf, buf)
            copies.append(_issue(buf))
        for buf in range(_NBUF):
            cp_a, cp_b = copies[buf]
            cp_a.wait()
            cp_b.wait()
            _compute(it * _NBUF + buf, buf)
        return carry

    lax.fori_loop(0, num_chunks // _NBUF, _outer, 0)

    pltpu.sync_copy(ni_v, ni_hbm.at[wid])


def _sc_edge_scores(xh, ei_pad, *, e_real, n_pad):
    n, c = xh.shape
    e_per_w = ei_pad.shape[1] // _NUM_SUBCORES
    mesh = plsc.VectorSubcoreMesh(
        core_axis_name="c", subcore_axis_name="s",
        num_cores=_NUM_CORES, num_subcores=_NUM_SUBCORES)
    scratch = (
        [pltpu.VMEM((_CHUNK,), jnp.int32)] * (2 * _NBUF)
        + [pltpu.VMEM((_CHUNK, c), jnp.int32)] * (2 * _NBUF)
        + [pltpu.VMEM((_LANES * _LANES,), jnp.float32),
           pltpu.VMEM((n_pad,), jnp.float32),
           pltpu.VMEM_SHARED((n, c), jnp.int32)]
        + [pltpu.SemaphoreType.DMA] * (2 * _NBUF)
    )
    kfn = pl.kernel(
        functools.partial(_edge_score_body, e_real=e_real, e_per_w=e_per_w,
                          n_pad=n_pad, c_chunks=c // _LANES),
        out_type=jax.ShapeDtypeStruct((_NW, n_pad), jnp.float32),
        mesh=mesh,
        compiler_params=pltpu.CompilerParams(needs_layout_passes=False,
                                             use_tc_tiling_on_sc=False),
        scratch_types=scratch,
    )
    return kfn(xh, ei_pad)


def _pool_body(xe_ref, xo_ref, nie_ref, nio_ref, out_ref):
    we = jax.nn.sigmoid(jnp.sum(nie_ref[...], axis=1, keepdims=True))
    wo = jax.nn.sigmoid(jnp.sum(nio_ref[...], axis=1, keepdims=True))
    out_ref[...] = 0.5 * (xe_ref[...] * we + xo_ref[...] * wo)


def _tc_pool(xe, xo, nie_t, nio_t):
    k, c = xe.shape
    blk = 1000
    grid = k // blk
    return pl.pallas_call(
        _pool_body,
        grid=(grid,),
        in_specs=[
            pl.BlockSpec((blk, c), lambda i: (i, 0)),
            pl.BlockSpec((blk, c), lambda i: (i, 0)),
            pl.BlockSpec((blk, _NW), lambda i: (i, 0)),
            pl.BlockSpec((blk, _NW), lambda i: (i, 0)),
        ],
        out_specs=pl.BlockSpec((blk, c), lambda i: (i, 0)),
        out_shape=jax.ShapeDtypeStruct((k, c), jnp.float32),
    )(xe, xo, nie_t, nio_t)


def kernel(x, edge_index):
    b, n, c = x.shape
    e = edge_index.shape[1]
    num_keep = max(1, int(n * 0.5))

    x2d = x.reshape(n, c)

    tile_e = _NW * _CHUNK * _NBUF
    e_pad = ((e + tile_e - 1) // tile_e) * tile_e
    ei_pad = jnp.pad(edge_index.astype(jnp.int32), ((0, 0), (0, e_pad - e)))
    n_pad = ((n + _LANES - 1) // _LANES) * _LANES

    xh = jax.lax.bitcast_convert_type(
        x2d.astype(jnp.bfloat16).reshape(n, c // 2, 2), jnp.int32)
    nip = _sc_edge_scores(xh, ei_pad, e_real=e, n_pad=n_pad)

    ni = nip[:, :n]
    nie_t = ni[:, 0::2].T
    nio_t = ni[:, 1::2].T
    xe = x2d[0::2]
    xo = x2d[1::2]

    pooled = _tc_pool(xe, xo, nie_t, nio_t)
    x_pooled = pooled.reshape(b, num_keep, c)

    idx = jnp.arange(num_keep)
    left = idx[:-1]
    right = idx[1:]
    new_edge_index = jnp.concatenate(
        [jnp.stack([left, right], axis=0), jnp.stack([right, left], axis=0)],
        axis=1)
    return x_pooled, new_edge_index


# speedup vs baseline: 1.4850x; 1.1504x over previous
import functools

import jax
import jax.numpy as jnp
from jax import lax
from jax.experimental import pallas as pl
from jax.experimental.pallas import tpu as pltpu
from jax.experimental.pallas import tpu_sc as plsc

_NUM_CORES = 2
_NUM_SUBCORES = 16
_NW = _NUM_CORES * _NUM_SUBCORES
_LANES = 16

_CHUNK = 128


def _sqrt16(a):
    ai = lax.bitcast_convert_type(a, jnp.int32)
    yi = jnp.int32(0x5F3759DF) - (ai >> 1)
    y = lax.bitcast_convert_type(yi, jnp.float32)
    for _ in range(2):
        y = y * (1.5 - 0.5 * a * y * y)
    return a * y


_NBUF = 4


def _edge_score_body(x_hbm, ei_hbm, ni_hbm, *refs,
                     e_real, e_per_w, n_pad, c_chunks):
    row_bufs = refs[0:_NBUF]
    col_bufs = refs[_NBUF:2 * _NBUF]
    a_bufs = refs[2 * _NBUF:3 * _NBUF]
    b_bufs = refs[3 * _NBUF:4 * _NBUF]
    t_v = refs[4 * _NBUF]
    ni_v = refs[4 * _NBUF + 1]
    xs_ref = refs[4 * _NBUF + 2]
    a_sems = refs[4 * _NBUF + 3:5 * _NBUF + 3]
    b_sems = refs[5 * _NBUF + 3:6 * _NBUF + 3]
    sid = lax.axis_index("s")
    cid = lax.axis_index("c")
    wid = sid * _NUM_CORES + cid
    pair_chunks = e_per_w // _CHUNK
    c0 = max(_NBUF, (int(pair_chunks * 0.50) // _NBUF) * _NBUF)
    c1 = pair_chunks - c0
    num_chunks = jnp.where(cid == 0, c0, c1)
    base = sid * e_per_w + cid * (c0 * _CHUNK)

    zeros16 = jnp.zeros((_LANES,), jnp.float32)

    def _zero(i, carry):
        ni_v[pl.ds(i * _LANES, _LANES)] = zeros16
        return carry

    @pl.when(sid == 0)
    def _fill_shared():
        pltpu.sync_copy(x_hbm, xs_ref)

    plsc.subcore_barrier()

    lax.fori_loop(0, n_pad // _LANES, _zero, 0)

    def _load_idx(ch, buf):
        off = base + ch * _CHUNK
        pltpu.sync_copy(ei_hbm.at[0, pl.ds(off, _CHUNK)], row_bufs[buf])
        pltpu.sync_copy(ei_hbm.at[1, pl.ds(off, _CHUNK)], col_bufs[buf])

    def _issue(buf):
        cp_a = pltpu.async_copy(xs_ref.at[row_bufs[buf]], a_bufs[buf],
                                a_sems[buf])
        cp_b = pltpu.async_copy(xs_ref.at[col_bufs[buf]], b_bufs[buf],
                                b_sems[buf])
        return cp_a, cp_b

    def _compute(ch, buf):
        a_v = a_bufs[buf]
        b_v = b_bufs[buf]
        off = base + ch * _CHUNK

        def _group_body(g, gcarry):
            gbase = g * _LANES
            lanes = lax.iota(jnp.int32, _LANES)
            for e in range(_LANES):
                r = gbase + e
                acc = jnp.zeros((2 * _LANES,), jnp.bfloat16)
                for c in range(c_chunks):
                    aw = a_v[r, pl.ds(c * _LANES, _LANES)]
                    bw = b_v[r, pl.ds(c * _LANES, _LANES)]
                    av = plsc.bitcast(aw, jnp.bfloat16)
                    bv = plsc.bitcast(bw, jnp.bfloat16)
                    d = av - bv
                    acc = acc + d * d
                lo, hi = plsc.unpack(acc, format=plsc.PackFormat.INTERLEAVED)
                t_v[pl.ds(e * _LANES, _LANES)] = lo + hi
            colidx = lanes * _LANES
            dsq = jnp.zeros((_LANES,), jnp.float32)
            for c in range(_LANES):
                dsq = dsq + plsc.load_gather(t_v, [colidx + c])
            dist = _sqrt16(dsq + 1e-6)
            gid = off + gbase + lanes
            mask = gid < e_real
            ridx = row_bufs[buf][pl.ds(gbase, _LANES)]
            plsc.addupdate_scatter(ni_v, [ridx], dist, mask=mask)
            return gcarry

        lax.fori_loop(0, _CHUNK // _LANES, _group_body, 0)

    def _outer(it, carry):
        copies = []
        for buf in range(_NBUF):
            _load_idx(it * _NBUF + buf, buf)
            copies.append(_issue(buf))
        for buf in range(_NBUF):
            cp_a, cp_b = copies[buf]
            cp_a.wait()
            cp_b.wait()
            _compute(it * _NBUF + buf, buf)
        return carry

    lax.fori_loop(0, num_chunks // _NBUF, _outer, 0)

    pltpu.sync_copy(ni_v, ni_hbm.at[wid])


def _sc_edge_scores(xh, ei_pad, *, e_real, n_pad):
    n, c = xh.shape
    e_per_w = ei_pad.shape[1] // _NUM_SUBCORES
    mesh = plsc.VectorSubcoreMesh(
        core_axis_name="c", subcore_axis_name="s",
        num_cores=_NUM_CORES, num_subcores=_NUM_SUBCORES)
    scratch = (
        [pltpu.VMEM((_CHUNK,), jnp.int32)] * (2 * _NBUF)
        + [pltpu.VMEM((_CHUNK, c), jnp.int32)] * (2 * _NBUF)
        + [pltpu.VMEM((_LANES * _LANES,), jnp.float32),
           pltpu.VMEM((n_pad,), jnp.float32),
           pltpu.VMEM_SHARED((n, c), jnp.int32)]
        + [pltpu.SemaphoreType.DMA] * (2 * _NBUF)
    )
    kfn = pl.kernel(
        functools.partial(_edge_score_body, e_real=e_real, e_per_w=e_per_w,
                          n_pad=n_pad, c_chunks=c // _LANES),
        out_type=jax.ShapeDtypeStruct((_NW, n_pad), jnp.float32),
        mesh=mesh,
        compiler_params=pltpu.CompilerParams(needs_layout_passes=False,
                                             use_tc_tiling_on_sc=False),
        scratch_types=scratch,
    )
    return kfn(xh, ei_pad)


def _pool_body(xe_ref, xo_ref, nie_ref, nio_ref, out_ref):
    we = jax.nn.sigmoid(jnp.sum(nie_ref[...], axis=1, keepdims=True))
    wo = jax.nn.sigmoid(jnp.sum(nio_ref[...], axis=1, keepdims=True))
    out_ref[...] = 0.5 * (xe_ref[...] * we + xo_ref[...] * wo)


def _tc_pool(xe, xo, nie_t, nio_t):
    k, c = xe.shape
    blk = 1000
    grid = k // blk
    return pl.pallas_call(
        _pool_body,
        grid=(grid,),
        in_specs=[
            pl.BlockSpec((blk, c), lambda i: (i, 0)),
            pl.BlockSpec((blk, c), lambda i: (i, 0)),
            pl.BlockSpec((blk, _NW), lambda i: (i, 0)),
            pl.BlockSpec((blk, _NW), lambda i: (i, 0)),
        ],
        out_specs=pl.BlockSpec((blk, c), lambda i: (i, 0)),
        out_shape=jax.ShapeDtypeStruct((k, c), jnp.float32),
    )(xe, xo, nie_t, nio_t)


def kernel(x, edge_index):
    b, n, c = x.shape
    e = edge_index.shape[1]
    num_keep = max(1, int(n * 0.5))

    x2d = x.reshape(n, c)

    tile_e = _NW * _CHUNK * _NBUF
    e_pad = ((e + tile_e - 1) // tile_e) * tile_e
    ei_pad = jnp.pad(edge_index.astype(jnp.int32), ((0, 0), (0, e_pad - e)))
    n_pad = ((n + _LANES - 1) // _LANES) * _LANES

    xh = jax.lax.bitcast_convert_type(
        x2d.astype(jnp.bfloat16).reshape(n, c // 2, 2), jnp.int32)
    nip = _sc_edge_scores(xh, ei_pad, e_real=e, n_pad=n_pad)

    ni = nip[:, :n]
    nie_t = ni[:, 0::2].T
    nio_t = ni[:, 1::2].T
    xe = x2d[0::2]
    xo = x2d[1::2]

    pooled = _tc_pool(xe, xo, nie_t, nio_t)
    x_pooled = pooled.reshape(b, num_keep, c)

    idx = jnp.arange(num_keep)
    left = idx[:-1]
    right = idx[1:]
    new_edge_index = jnp.concatenate(
        [jnp.stack([left, right], axis=0), jnp.stack([right, left], axis=0)],
        axis=1)
    return x_pooled, new_edge_index
